# Initial kernel scaffold; baseline (speedup 1.0000x reference)
import jax
import jax.numpy as jnp
from jax.experimental import pallas as pl


def kernel(H, R, edge_index, edge_type, Wq_w, Wq_b, Wk_w, Wk_b, Wv_w, Wv_b, att_w, att_b, Wrel_w, Wrel_b, out_w, out_b, ln_g, ln_b):
    raise NotImplementedError("write your pallas kernel here")


# speedup vs baseline: 9.3872x; 9.3872x over previous
import functools

import jax
import jax.numpy as jnp
from jax import lax
from jax.experimental import pallas as pl
from jax.experimental.pallas import tpu as pltpu
from jax.experimental.pallas import tpu_sc as plsc

N = 10000
E = 320000
DIM = 128
HEADS = 8
HD = DIM // HEADS
NREL = 16

NC = 2
NS = 16
L = 16
NW = NC * NS
EW = E // NW
CHUNK = 80
NCHUNK = EW // CHUNK
N_PAD = 10240
ROWS_PER_SUB = N_PAD // NS

BN = 2000


def _k1_body(h_ref, r_ref, wq, bq, wk, bk, wv, bv, wrel, brel, attf,
             qp_o, k_o, v_o, rrel_o):
    h = h_ref[...]
    q = jnp.dot(h, wq[...], preferred_element_type=jnp.float32) + bq[...]
    qp_o[...] = q * attf[...]
    k_o[...] = jnp.dot(h, wk[...], preferred_element_type=jnp.float32) + bk[...]
    v_o[...] = jnp.dot(h, wv[...], preferred_element_type=jnp.float32) + bv[...]

    @pl.when(pl.program_id(0) == 0)
    def _():
        rrel_o[...] = (jnp.dot(r_ref[...], wrel[...],
                               preferred_element_type=jnp.float32) + brel[...])


def _k1(H, R, Wq_w, Wq_b, Wk_w, Wk_b, Wv_w, Wv_b, Wrel_w, Wrel_b, att_full):
    full = pl.BlockSpec((DIM, DIM), lambda i: (0, 0))
    vec = pl.BlockSpec((1, DIM), lambda i: (0, 0))
    row = pl.BlockSpec((BN, DIM), lambda i: (i, 0))
    rfull = pl.BlockSpec((NREL, DIM), lambda i: (0, 0))
    out_sh = jax.ShapeDtypeStruct((N, DIM), jnp.float32)
    return pl.pallas_call(
        _k1_body,
        grid=(N // BN,),
        in_specs=[row, rfull, full, vec, full, vec, full, vec, full, vec, vec],
        out_specs=[row, row, row, rfull],
        out_shape=[out_sh, out_sh, out_sh,
                   jax.ShapeDtypeStruct((NREL, DIM), jnp.float32)],
    )(H, R, Wq_w, Wq_b.reshape(1, DIM), Wk_w, Wk_b.reshape(1, DIM),
      Wv_w, Wv_b.reshape(1, DIM), Wrel_w, Wrel_b.reshape(1, DIM),
      att_full.reshape(1, DIM))


def _k2_body(qp_hbm, k_hbm, v_hbm, rrel_hbm, src_hbm, dst_hbm, et_hbm,
             attb_hbm, zeros_hbm, outp_hbm, attn_hbm,
             src_v, dst_v, et_v, qp_rows, k_rows, v_rows, rrel_v,
             attn_v, attb_v, acc, sem):
    cid = lax.axis_index("c")
    sid = lax.axis_index("s")
    wid = sid * NC + cid

    pltpu.sync_copy(zeros_hbm.at[pl.ds(sid * ROWS_PER_SUB, ROWS_PER_SUB)],
                    acc.at[pl.ds(sid * ROWS_PER_SUB, ROWS_PER_SUB)])
    pltpu.sync_copy(attb_hbm, attb_v)
    pltpu.sync_copy(rrel_hbm, rrel_v)
    plsc.subcore_barrier()

    def chunk_body(ci, _):
        base = wid * EW + ci * CHUNK
        pltpu.sync_copy(src_hbm.at[pl.ds(base, CHUNK)], src_v)
        pltpu.sync_copy(dst_hbm.at[pl.ds(base, CHUNK)], dst_v)
        pltpu.sync_copy(et_hbm.at[pl.ds(base, CHUNK)], et_v)
        cp1 = pltpu.async_copy(qp_hbm.at[dst_v], qp_rows, sem)
        cp2 = pltpu.async_copy(k_hbm.at[src_v], k_rows, sem)
        cp3 = pltpu.async_copy(v_hbm.at[src_v], v_rows, sem)
        cp1.wait()
        cp2.wait()
        cp3.wait()
        attb = attb_v[...]

        def group_body(g, _):
            rows = jnp.arange(L, dtype=jnp.int32) + g * L
            et_g = et_v[pl.ds(g * L, L)]
            wsum = jnp.zeros((L,), jnp.float32)
            ws = []
            for h in range(HEADS):
                s = attb
                for d in range(HD):
                    cols = jnp.full((L,), h * HD + d, dtype=jnp.int32)
                    qp = plsc.load_gather(qp_rows, [rows, cols])
                    kk = plsc.load_gather(k_rows, [rows, cols])
                    rr = plsc.load_gather(rrel_v, [et_g, cols])
                    s = s + qp * (kk + rr)
                w = 1.0 / (1.0 + jnp.exp(-s))
                ws.append(w)
                wsum = wsum + w
            attn_v[pl.ds(g * L, L)] = wsum * (1.0 / HEADS)
            for h in range(HEADS):
                for d in range(HD):
                    cols = jnp.full((L,), h * HD + d, dtype=jnp.int32)
                    vv = plsc.load_gather(v_rows, [rows, cols])
                    rr = plsc.load_gather(rrel_v, [et_g, cols])
                    plsc.store_scatter(qp_rows, [rows, cols],
                                       ws[h] * (vv + rr))
            return 0

        lax.fori_loop(0, CHUNK // L, group_body, 0)
        pltpu.sync_copy(qp_rows, acc.at[dst_v], add=True)
        pltpu.sync_copy(attn_v, attn_hbm.at[pl.ds(base, CHUNK)])
        return 0

    lax.fori_loop(0, NCHUNK, chunk_body, 0)
    plsc.subcore_barrier()
    pltpu.sync_copy(acc.at[pl.ds(sid * ROWS_PER_SUB, ROWS_PER_SUB)],
                    outp_hbm.at[cid, pl.ds(sid * ROWS_PER_SUB, ROWS_PER_SUB)])


def _k2(Qp, K, V, Rrel, src, dst, et, attb16, zeros):
    mesh = plsc.VectorSubcoreMesh(core_axis_name="c", subcore_axis_name="s")
    f = pl.kernel(
        _k2_body,
        out_type=(jax.ShapeDtypeStruct((NC, N_PAD, DIM), jnp.float32),
                  jax.ShapeDtypeStruct((E,), jnp.float32)),
        mesh=mesh,
        scratch_types=[
            pltpu.VMEM((CHUNK,), jnp.int32),
            pltpu.VMEM((CHUNK,), jnp.int32),
            pltpu.VMEM((CHUNK,), jnp.int32),
            pltpu.VMEM((CHUNK, DIM), jnp.float32),
            pltpu.VMEM((CHUNK, DIM), jnp.float32),
            pltpu.VMEM((CHUNK, DIM), jnp.float32),
            pltpu.VMEM((NREL, DIM), jnp.float32),
            pltpu.VMEM((CHUNK,), jnp.float32),
            pltpu.VMEM((L,), jnp.float32),
            pltpu.VMEM_SHARED((N_PAD, DIM), jnp.float32),
            pltpu.SemaphoreType.DMA,
        ],
        compiler_params=pltpu.CompilerParams(needs_layout_passes=False),
    )
    return f(Qp, K, V, Rrel, src, dst, et, attb16, zeros)


def _k3_body(op_ref, h_ref, w_ref, b_ref, g_ref, b2_ref, out_ref):
    o = op_ref[0] + op_ref[1]
    out = jnp.dot(o, w_ref[...], preferred_element_type=jnp.float32) + b_ref[...]
    act = jnp.where(out >= 0, out, 0.2 * out)
    x = h_ref[...] + act
    mu = jnp.mean(x, axis=1, keepdims=True)
    var = jnp.mean((x - mu) ** 2, axis=1, keepdims=True)
    out_ref[...] = (x - mu) / jnp.sqrt(var + 1e-5) * g_ref[...] + b2_ref[...]


def _k3(outp, H, out_w, out_b, ln_g, ln_b):
    full = pl.BlockSpec((DIM, DIM), lambda i: (0, 0))
    vec = pl.BlockSpec((1, DIM), lambda i: (0, 0))
    row = pl.BlockSpec((BN, DIM), lambda i: (i, 0))
    prow = pl.BlockSpec((NC, BN, DIM), lambda i: (0, i, 0))
    return pl.pallas_call(
        _k3_body,
        grid=(N // BN,),
        in_specs=[prow, row, full, vec, vec, vec],
        out_specs=row,
        out_shape=jax.ShapeDtypeStruct((N, DIM), jnp.float32),
    )(outp, H, out_w, out_b.reshape(1, DIM), ln_g.reshape(1, DIM),
      ln_b.reshape(1, DIM))


def kernel(H, R, edge_index, edge_type, Wq_w, Wq_b, Wk_w, Wk_b, Wv_w, Wv_b,
           att_w, att_b, Wrel_w, Wrel_b, out_w, out_b, ln_g, ln_b):
    att_full = jnp.tile(att_w[:, 0], HEADS)
    Qp, K, V, Rrel = _k1(H, R, Wq_w, Wq_b, Wk_w, Wk_b, Wv_w, Wv_b,
                         Wrel_w, Wrel_b, att_full)
    src = edge_index[0]
    dst = edge_index[1]
    attb16 = jnp.broadcast_to(att_b, (L,)).astype(jnp.float32)
    zeros = jnp.zeros((N_PAD, DIM), jnp.float32)
    outp, attn_mean = _k2(Qp, K, V, Rrel, src, dst, edge_type, attb16, zeros)
    h_out = _k3(outp, H, out_w, out_b, ln_g, ln_b)
    return (h_out, attn_mean)


# speedup vs baseline: 15.8301x; 1.6864x over previous
import functools

import jax
import jax.numpy as jnp
from jax import lax
from jax.experimental import pallas as pl
from jax.experimental.pallas import tpu as pltpu
from jax.experimental.pallas import tpu_sc as plsc

N = 10000
E = 320000
DIM = 128
HEADS = 8
HD = DIM // HEADS
NREL = 16

NC = 2
NS = 16
L = 16
NW = NC * NS
EW = E // NW
CHUNK = 80
NCHUNK = EW // CHUNK
N_PAD = 10240
ROWS_PER_SUB = N_PAD // NS

BN = 2000


def _k1_body(h_ref, r_ref, wq, bq, wk, bk, wv, bv, wrel, brel, attf,
             qp_o, k_o, v_o, rrel_o, qr_o):
    h = h_ref[...]
    q = jnp.dot(h, wq[...], preferred_element_type=jnp.float32) + bq[...]
    qp = q * attf[...]
    qp_o[...] = qp
    k_o[...] = jnp.dot(h, wk[...], preferred_element_type=jnp.float32) + bk[...]
    v_o[...] = jnp.dot(h, wv[...], preferred_element_type=jnp.float32) + bv[...]
    rrel = (jnp.dot(r_ref[...], wrel[...],
                    preferred_element_type=jnp.float32) + brel[...])
    qrs = []
    for h_i in range(HEADS):
        qp_h = qp[:, h_i * HD:(h_i + 1) * HD]
        rr_h = rrel[:, h_i * HD:(h_i + 1) * HD]
        qrs.append(jax.lax.dot_general(
            qp_h, rr_h, (((1,), (1,)), ((), ())),
            preferred_element_type=jnp.float32))
    qr_o[...] = jnp.concatenate(qrs, axis=1)

    @pl.when(pl.program_id(0) == 0)
    def _():
        rrel_o[...] = rrel


def _k1(H, R, Wq_w, Wq_b, Wk_w, Wk_b, Wv_w, Wv_b, Wrel_w, Wrel_b, att_full):
    full = pl.BlockSpec((DIM, DIM), lambda i: (0, 0))
    vec = pl.BlockSpec((1, DIM), lambda i: (0, 0))
    row = pl.BlockSpec((BN, DIM), lambda i: (i, 0))
    rfull = pl.BlockSpec((NREL, DIM), lambda i: (0, 0))
    out_sh = jax.ShapeDtypeStruct((N, DIM), jnp.float32)
    return pl.pallas_call(
        _k1_body,
        grid=(N // BN,),
        in_specs=[row, rfull, full, vec, full, vec, full, vec, full, vec, vec],
        out_specs=[row, row, row, rfull, row],
        out_shape=[out_sh, out_sh, out_sh,
                   jax.ShapeDtypeStruct((NREL, DIM), jnp.float32), out_sh],
    )(H, R, Wq_w, Wq_b.reshape(1, DIM), Wk_w, Wk_b.reshape(1, DIM),
      Wv_w, Wv_b.reshape(1, DIM), Wrel_w, Wrel_b.reshape(1, DIM),
      att_full.reshape(1, DIM))


def _k2_body(qp_hbm, kv_hbm, rrel_hbm, src_hbm, dst_hbm, et_hbm,
             attb_hbm, zeros_hbm, outp_hbm, attn_hbm,
             src_v, dst_v, et_v, qp_rows, kv_rows, rrel_v, msg_rows,
             attn_v, attb_v, acc, sem):
    cid = lax.axis_index("c")
    sid = lax.axis_index("s")
    wid = sid * NC + cid

    pltpu.sync_copy(zeros_hbm.at[pl.ds(sid * ROWS_PER_SUB, ROWS_PER_SUB)],
                    acc.at[pl.ds(sid * ROWS_PER_SUB, ROWS_PER_SUB)])
    pltpu.sync_copy(attb_hbm, attb_v)
    pltpu.sync_copy(rrel_hbm, rrel_v)
    plsc.subcore_barrier()

    def chunk_body(ci, _):
        base = wid * EW + ci * CHUNK
        pltpu.sync_copy(src_hbm.at[pl.ds(base, CHUNK)], src_v)
        pltpu.sync_copy(dst_hbm.at[pl.ds(base, CHUNK)], dst_v)
        pltpu.sync_copy(et_hbm.at[pl.ds(base, CHUNK)], et_v)
        cp1 = pltpu.async_copy(qp_hbm.at[dst_v], qp_rows, sem)
        cp2 = pltpu.async_copy(kv_hbm.at[src_v], kv_rows, sem)
        cp1.wait()
        cp2.wait()
        attb = attb_v[...]

        def _unpack2(x):
            return plsc.unpack(plsc.bitcast(x, jnp.bfloat16),
                               format=plsc.PackFormat.INTERLEAVED,
                               preferred_element_type=jnp.float32)

        HP = HD // 2

        def group_body(g, _):
            rows = jnp.arange(L, dtype=jnp.int32) + g * L
            et_g = et_v[pl.ds(g * L, L)]
            et_par = jnp.equal(et_g & 1, 0)
            wsum = jnp.zeros((L,), jnp.float32)
            ws = []
            for h in range(HEADS):
                qre, qro = _unpack2(plsc.load_gather(
                    qp_rows, [rows, (et_g >> 1) + (DIM // 2 + h * (NREL // 2))]))
                s = attb + jnp.where(et_par, qre, qro)
                for dp in range(HP):
                    cols = jnp.full((L,), h * HP + dp, dtype=jnp.int32)
                    qe, qo = _unpack2(plsc.load_gather(qp_rows, [rows, cols]))
                    ke, ko = _unpack2(plsc.load_gather(kv_rows, [rows, cols]))
                    s = s + qe * ke + qo * ko
                w = 1.0 / (1.0 + jnp.exp(-s))
                ws.append(w)
                wsum = wsum + w
            attn_v[pl.ds(g * L, L)] = wsum * (1.0 / HEADS)
            for h in range(HEADS):
                for dp in range(HP):
                    cols = jnp.full((L,), h * HP + dp, dtype=jnp.int32)
                    ve, vo = _unpack2(
                        plsc.load_gather(kv_rows, [rows, cols + (DIM // 2)]))
                    re, ro = _unpack2(plsc.load_gather(rrel_v, [et_g, cols]))
                    ce = jnp.full((L,), h * HD + 2 * dp, dtype=jnp.int32)
                    plsc.store_scatter(msg_rows, [rows, ce], ws[h] * (ve + re))
                    plsc.store_scatter(msg_rows, [rows, ce + 1],
                                       ws[h] * (vo + ro))
            return 0

        lax.fori_loop(0, CHUNK // L, group_body, 0)
        pltpu.sync_copy(msg_rows, acc.at[dst_v], add=True)
        pltpu.sync_copy(attn_v, attn_hbm.at[pl.ds(base, CHUNK)])
        return 0

    lax.fori_loop(0, NCHUNK, chunk_body, 0)
    plsc.subcore_barrier()
    pltpu.sync_copy(acc.at[pl.ds(sid * ROWS_PER_SUB, ROWS_PER_SUB)],
                    outp_hbm.at[cid, pl.ds(sid * ROWS_PER_SUB, ROWS_PER_SUB)])


def _k2(Qp_b, KV, Rrel_p, src, dst, et, attb16, zeros):
    mesh = plsc.VectorSubcoreMesh(core_axis_name="c", subcore_axis_name="s")
    f = pl.kernel(
        _k2_body,
        out_type=(jax.ShapeDtypeStruct((NC, N_PAD, DIM), jnp.float32),
                  jax.ShapeDtypeStruct((E,), jnp.float32)),
        mesh=mesh,
        scratch_types=[
            pltpu.VMEM((CHUNK,), jnp.int32),
            pltpu.VMEM((CHUNK,), jnp.int32),
            pltpu.VMEM((CHUNK,), jnp.int32),
            pltpu.VMEM((CHUNK, DIM), jnp.int32),
            pltpu.VMEM((CHUNK, DIM), jnp.int32),
            pltpu.VMEM((NREL, DIM // 2), jnp.int32),
            pltpu.VMEM((CHUNK, DIM), jnp.float32),
            pltpu.VMEM((CHUNK,), jnp.float32),
            pltpu.VMEM((L,), jnp.float32),
            pltpu.VMEM_SHARED((N_PAD, DIM), jnp.float32),
            pltpu.SemaphoreType.DMA,
        ],
        compiler_params=pltpu.CompilerParams(needs_layout_passes=False),
    )
    return f(Qp_b, KV, Rrel_p, src, dst, et, attb16, zeros)


def _k3_body(op_ref, h_ref, w_ref, b_ref, g_ref, b2_ref, out_ref):
    o = op_ref[0] + op_ref[1]
    out = jnp.dot(o, w_ref[...], preferred_element_type=jnp.float32) + b_ref[...]
    act = jnp.where(out >= 0, out, 0.2 * out)
    x = h_ref[...] + act
    mu = jnp.mean(x, axis=1, keepdims=True)
    var = jnp.mean((x - mu) ** 2, axis=1, keepdims=True)
    out_ref[...] = (x - mu) / jnp.sqrt(var + 1e-5) * g_ref[...] + b2_ref[...]


def _k3(outp, H, out_w, out_b, ln_g, ln_b):
    full = pl.BlockSpec((DIM, DIM), lambda i: (0, 0))
    vec = pl.BlockSpec((1, DIM), lambda i: (0, 0))
    row = pl.BlockSpec((BN, DIM), lambda i: (i, 0))
    prow = pl.BlockSpec((NC, BN, DIM), lambda i: (0, i, 0))
    return pl.pallas_call(
        _k3_body,
        grid=(N // BN,),
        in_specs=[prow, row, full, vec, vec, vec],
        out_specs=row,
        out_shape=jax.ShapeDtypeStruct((N, DIM), jnp.float32),
    )(outp, H, out_w, out_b.reshape(1, DIM), ln_g.reshape(1, DIM),
      ln_b.reshape(1, DIM))


def kernel(H, R, edge_index, edge_type, Wq_w, Wq_b, Wk_w, Wk_b, Wv_w, Wv_b,
           att_w, att_b, Wrel_w, Wrel_b, out_w, out_b, ln_g, ln_b):
    att_full = jnp.tile(att_w[:, 0], HEADS)
    Qp, K, V, Rrel, QR = _k1(H, R, Wq_w, Wq_b, Wk_w, Wk_b, Wv_w, Wv_b,
                             Wrel_w, Wrel_b, att_full)

    def _pack(x):
        b = x.astype(jnp.bfloat16).reshape(x.shape[0], DIM // 2, 2)
        return jax.lax.bitcast_convert_type(b, jnp.int32)

    src = edge_index[0]
    dst = edge_index[1]
    attb16 = jnp.broadcast_to(att_b, (L,)).astype(jnp.float32)
    zeros = jnp.zeros((N_PAD, DIM), jnp.float32)
    KV = jnp.concatenate([_pack(K), _pack(V)], axis=1)
    QTAB = jnp.concatenate([_pack(Qp), _pack(QR)], axis=1)
    outp, attn_mean = _k2(QTAB, KV, _pack(Rrel),
                          src, dst, edge_type, attb16, zeros)
    h_out = _k3(outp, H, out_w, out_b, ln_g, ln_b)
    return (h_out, attn_mean)


# speedup vs baseline: 16.1044x; 1.0173x over previous
import functools

import jax
import jax.numpy as jnp
from jax import lax
from jax.experimental import pallas as pl
from jax.experimental.pallas import tpu as pltpu
from jax.experimental.pallas import tpu_sc as plsc

N = 10000
E = 320000
DIM = 128
HEADS = 8
HD = DIM // HEADS
HP = HD // 2
NREL = 16

NC = 2
NS = 16
L = 16
NW = NC * NS
EW = E // NW
CHUNK = 48
EW_PAD = 10080
E_PAD = EW_PAD * NW
NCHUNK = EW_PAD // CHUNK
N_PAD = 10240
ROWS_PER_SUB = N_PAD // NS

BN = N_PAD // 5


def _k1_body(h_ref, r_ref, wq, bq, wk, bk, wv, bv, wrel, brel, attf,
             qp_o, k_o, v_o, rrel_o, qr_o):
    h = h_ref[...]
    q = jnp.dot(h, wq[...], preferred_element_type=jnp.float32) + bq[...]
    qp = q * attf[...]
    qp_o[...] = qp
    k_o[...] = jnp.dot(h, wk[...], preferred_element_type=jnp.float32) + bk[...]
    v_o[...] = jnp.dot(h, wv[...], preferred_element_type=jnp.float32) + bv[...]
    rrel = (jnp.dot(r_ref[...], wrel[...],
                    preferred_element_type=jnp.float32) + brel[...])
    qrs = []
    for h_i in range(HEADS):
        qp_h = qp[:, h_i * HD:(h_i + 1) * HD]
        rr_h = rrel[:, h_i * HD:(h_i + 1) * HD]
        qrs.append(jax.lax.dot_general(
            qp_h, rr_h, (((1,), (1,)), ((), ())),
            preferred_element_type=jnp.float32))
    qr_o[...] = jnp.concatenate(qrs, axis=1)

    @pl.when(pl.program_id(0) == 0)
    def _():
        rrel_o[...] = rrel


def _k1(H, R, Wq_w, Wq_b, Wk_w, Wk_b, Wv_w, Wv_b, Wrel_w, Wrel_b, att_full):
    full = pl.BlockSpec((DIM, DIM), lambda i: (0, 0))
    vec = pl.BlockSpec((1, DIM), lambda i: (0, 0))
    row = pl.BlockSpec((BN, DIM), lambda i: (i, 0))
    rfull = pl.BlockSpec((NREL, DIM), lambda i: (0, 0))
    out_sh = jax.ShapeDtypeStruct((N_PAD, DIM), jnp.float32)
    return pl.pallas_call(
        _k1_body,
        grid=(N_PAD // BN,),
        in_specs=[row, rfull, full, vec, full, vec, full, vec, full, vec, vec],
        out_specs=[row, row, row, rfull, row],
        out_shape=[out_sh, out_sh, out_sh,
                   jax.ShapeDtypeStruct((NREL, DIM), jnp.float32), out_sh],
    )(H, R, Wq_w, Wq_b.reshape(1, DIM), Wk_w, Wk_b.reshape(1, DIM),
      Wv_w, Wv_b.reshape(1, DIM), Wrel_w, Wrel_b.reshape(1, DIM),
      att_full.reshape(1, DIM))


def _unpack2(x):
    return plsc.unpack(plsc.bitcast(x, jnp.bfloat16),
                       format=plsc.PackFormat.INTERLEAVED,
                       preferred_element_type=jnp.float32)


def _k2_body(qp_hbm, kv_hbm, rrel_hbm, src_hbm, dst_hbm, et_hbm,
             attb_hbm, zeros_hbm, outp_hbm, attn_hbm,
             src0, dst0, et0, src1, dst1, et1, qp0, qp1, kv0, kv1,
             msg_rows, attn_v, rrel_v, attb_v, acc,
             sem_g0, sem_g1, sem_i0, sem_i1):
    cid = lax.axis_index("c")
    sid = lax.axis_index("s")
    wid = sid * NC + cid
    ebase = wid * EW_PAD

    pltpu.sync_copy(zeros_hbm.at[pl.ds(sid * ROWS_PER_SUB, ROWS_PER_SUB)],
                    acc.at[pl.ds(sid * ROWS_PER_SUB, ROWS_PER_SUB)])
    pltpu.sync_copy(attb_hbm, attb_v)
    pltpu.sync_copy(rrel_hbm, rrel_v)
    plsc.subcore_barrier()
    attb = attb_v[...]

    def fire_idx(ci, src_v, dst_v, et_v, sem):
        b = ebase + ci * CHUNK
        pltpu.async_copy(src_hbm.at[pl.ds(b, CHUNK)], src_v, sem)
        pltpu.async_copy(dst_hbm.at[pl.ds(b, CHUNK)], dst_v, sem)
        pltpu.async_copy(et_hbm.at[pl.ds(b, CHUNK)], et_v, sem)

    def wait_idx(src_v, dst_v, et_v, sem):
        pltpu.make_async_copy(src_hbm.at[pl.ds(0, CHUNK)], src_v, sem).wait()
        pltpu.make_async_copy(dst_hbm.at[pl.ds(0, CHUNK)], dst_v, sem).wait()
        pltpu.make_async_copy(et_hbm.at[pl.ds(0, CHUNK)], et_v, sem).wait()

    def fire_gathers(src_v, dst_v, qp_rows, kv_rows, sem):
        pltpu.async_copy(qp_hbm.at[dst_v], qp_rows, sem)
        pltpu.async_copy(kv_hbm.at[src_v], kv_rows, sem)

    def wait_gathers(src_v, dst_v, qp_rows, kv_rows, sem):
        pltpu.make_async_copy(qp_hbm.at[dst_v], qp_rows, sem).wait()
        pltpu.make_async_copy(kv_hbm.at[src_v], kv_rows, sem).wait()

    def compute(ci, dst_v, et_ref, qp_rows, kv_rows):
        def group_body(g, _):
            rows = jnp.arange(L, dtype=jnp.int32) + g * L
            et_g = et_ref[pl.ds(g * L, L)]
            et_par = jnp.equal(et_g & 1, 0)
            qr_col = (et_g >> 1) + (DIM // 2)
            wsum = jnp.zeros((L,), jnp.float32)
            ws = []
            for h in range(HEADS):
                qre, qro = _unpack2(plsc.load_gather(
                    qp_rows, [rows, qr_col + h * (NREL // 2)]))
                s = attb + jnp.where(et_par, qre, qro)
                for dp in range(HP):
                    cols = jnp.full((L,), h * HP + dp, dtype=jnp.int32)
                    qe, qo = _unpack2(plsc.load_gather(qp_rows, [rows, cols]))
                    ke, ko = _unpack2(plsc.load_gather(kv_rows, [rows, cols]))
                    s = s + qe * ke + qo * ko
                w = 1.0 / (1.0 + jnp.exp(-s))
                ws.append(w)
                wsum = wsum + w
            attn_v[pl.ds(ci * CHUNK + g * L, L)] = wsum * (1.0 / HEADS)
            for h in range(HEADS):
                for dp in range(HP):
                    cols = jnp.full((L,), h * HP + dp, dtype=jnp.int32)
                    ve, vo = _unpack2(
                        plsc.load_gather(kv_rows, [rows, cols + (DIM // 2)]))
                    re, ro = _unpack2(plsc.load_gather(rrel_v, [et_g, cols]))
                    ce = jnp.full((L,), h * HD + 2 * dp, dtype=jnp.int32)
                    plsc.store_scatter(msg_rows, [rows, ce], ws[h] * (ve + re))
                    plsc.store_scatter(msg_rows, [rows, ce + 1],
                                       ws[h] * (vo + ro))
            return 0

        lax.fori_loop(0, CHUNK // L, group_body, 0)
        pltpu.sync_copy(msg_rows, acc.at[dst_v], add=True)

    fire_idx(0, src0, dst0, et0, sem_i0)
    wait_idx(src0, dst0, et0, sem_i0)
    fire_gathers(src0, dst0, qp0, kv0, sem_g0)
    fire_idx(1, src1, dst1, et1, sem_i1)

    def pair_body(ci2, _):
        ci = ci2 * 2
        wait_idx(src1, dst1, et1, sem_i1)
        fire_gathers(src1, dst1, qp1, kv1, sem_g1)
        wait_gathers(src0, dst0, qp0, kv0, sem_g0)
        compute(ci, dst0, et0, qp0, kv0)
        fire_idx(jnp.minimum(ci + 2, NCHUNK - 1), src0, dst0, et0, sem_i0)
        wait_idx(src0, dst0, et0, sem_i0)
        fire_gathers(src0, dst0, qp0, kv0, sem_g0)
        wait_gathers(src1, dst1, qp1, kv1, sem_g1)
        compute(ci + 1, dst1, et1, qp1, kv1)
        fire_idx(jnp.minimum(ci + 3, NCHUNK - 1), src1, dst1, et1, sem_i1)
        return 0

    lax.fori_loop(0, NCHUNK // 2, pair_body, 0)
    wait_idx(src1, dst1, et1, sem_i1)
    wait_gathers(src0, dst0, qp0, kv0, sem_g0)
    pltpu.sync_copy(attn_v, attn_hbm.at[pl.ds(ebase, EW_PAD)])
    plsc.subcore_barrier()
    pltpu.sync_copy(acc.at[pl.ds(sid * ROWS_PER_SUB, ROWS_PER_SUB)],
                    outp_hbm.at[cid, pl.ds(sid * ROWS_PER_SUB, ROWS_PER_SUB)])


def _k2(QTAB, KV, Rrel_p, src, dst, et, attb16, zeros):
    mesh = plsc.VectorSubcoreMesh(core_axis_name="c", subcore_axis_name="s")
    f = pl.kernel(
        _k2_body,
        out_type=(jax.ShapeDtypeStruct((NC, N_PAD, DIM), jnp.float32),
                  jax.ShapeDtypeStruct((E_PAD,), jnp.float32)),
        mesh=mesh,
        scratch_types=[
            pltpu.VMEM((CHUNK,), jnp.int32),
            pltpu.VMEM((CHUNK,), jnp.int32),
            pltpu.VMEM((CHUNK,), jnp.int32),
            pltpu.VMEM((CHUNK,), jnp.int32),
            pltpu.VMEM((CHUNK,), jnp.int32),
            pltpu.VMEM((CHUNK,), jnp.int32),
            pltpu.VMEM((CHUNK, DIM), jnp.int32),
            pltpu.VMEM((CHUNK, DIM), jnp.int32),
            pltpu.VMEM((CHUNK, DIM), jnp.int32),
            pltpu.VMEM((CHUNK, DIM), jnp.int32),
            pltpu.VMEM((CHUNK, DIM), jnp.float32),
            pltpu.VMEM((EW_PAD,), jnp.float32),
            pltpu.VMEM((NREL, DIM // 2), jnp.int32),
            pltpu.VMEM((L,), jnp.float32),
            pltpu.VMEM_SHARED((N_PAD, DIM), jnp.float32),
            pltpu.SemaphoreType.DMA,
            pltpu.SemaphoreType.DMA,
            pltpu.SemaphoreType.DMA,
            pltpu.SemaphoreType.DMA,
        ],
        compiler_params=pltpu.CompilerParams(needs_layout_passes=False),
    )
    return f(QTAB, KV, Rrel_p, src, dst, et, attb16, zeros)


def _k3_body(op_ref, h_ref, w_ref, b_ref, g_ref, b2_ref, out_ref):
    o = op_ref[0] + op_ref[1]
    out = jnp.dot(o, w_ref[...], preferred_element_type=jnp.float32) + b_ref[...]
    act = jnp.where(out >= 0, out, 0.2 * out)
    x = h_ref[...] + act
    mu = jnp.mean(x, axis=1, keepdims=True)
    var = jnp.mean((x - mu) ** 2, axis=1, keepdims=True)
    out_ref[...] = (x - mu) / jnp.sqrt(var + 1e-5) * g_ref[...] + b2_ref[...]


def _k3(outp, H, out_w, out_b, ln_g, ln_b):
    full = pl.BlockSpec((DIM, DIM), lambda i: (0, 0))
    vec = pl.BlockSpec((1, DIM), lambda i: (0, 0))
    row = pl.BlockSpec((BN, DIM), lambda i: (i, 0))
    prow = pl.BlockSpec((NC, BN, DIM), lambda i: (0, i, 0))
    return pl.pallas_call(
        _k3_body,
        grid=(N_PAD // BN,),
        in_specs=[prow, row, full, vec, vec, vec],
        out_specs=row,
        out_shape=jax.ShapeDtypeStruct((N_PAD, DIM), jnp.float32),
    )(outp, H, out_w, out_b.reshape(1, DIM), ln_g.reshape(1, DIM),
      ln_b.reshape(1, DIM))


def kernel(H, R, edge_index, edge_type, Wq_w, Wq_b, Wk_w, Wk_b, Wv_w, Wv_b,
           att_w, att_b, Wrel_w, Wrel_b, out_w, out_b, ln_g, ln_b):
    att_full = jnp.tile(att_w[:, 0], HEADS)
    H_pad = jnp.pad(H, ((0, N_PAD - N), (0, 0)))
    Qp, K, V, Rrel, QR = _k1(H_pad, R, Wq_w, Wq_b, Wk_w, Wk_b, Wv_w, Wv_b,
                             Wrel_w, Wrel_b, att_full)

    def _pack(x):
        b = x.astype(jnp.bfloat16).reshape(x.shape[0], DIM // 2, 2)
        return jax.lax.bitcast_convert_type(b, jnp.int32)

    pad_spec = ((0, 0), (0, EW_PAD - EW))
    src_p = jnp.pad(edge_index[0].reshape(NW, EW), pad_spec).reshape(-1)
    dst_p = jnp.pad(edge_index[1].reshape(NW, EW), pad_spec,
                    constant_values=N_PAD - 1).reshape(-1)
    et_p = jnp.pad(edge_type.reshape(NW, EW), pad_spec).reshape(-1)

    attb16 = jnp.broadcast_to(att_b, (L,)).astype(jnp.float32)
    zeros = jnp.zeros((N_PAD, DIM), jnp.float32)
    KV = jnp.concatenate([_pack(K), _pack(V)], axis=1)
    QTAB = jnp.concatenate([_pack(Qp), _pack(QR)], axis=1)
    outp, attn_pad = _k2(QTAB, KV, _pack(Rrel), src_p, dst_p, et_p,
                         attb16, zeros)
    attn_mean = attn_pad.reshape(NW, EW_PAD)[:, :EW].reshape(E)
    h_out = _k3(outp, H_pad, out_w, out_b, ln_g, ln_b)[:N]
    return (h_out, attn_mean)


# speedup vs baseline: 17.1105x; 1.0625x over previous
import functools

import jax
import jax.numpy as jnp
from jax import lax
from jax.experimental import pallas as pl
from jax.experimental.pallas import tpu as pltpu
from jax.experimental.pallas import tpu_sc as plsc

N = 10000
E = 320000
DIM = 128
HEADS = 8
HD = DIM // HEADS
HP = HD // 2
NREL = 16

NC = 2
NS = 16
L = 16
NW = NC * NS
EW = E // NW
CHUNK = 48
EW_PAD = 10080
E_PAD = EW_PAD * NW
NCHUNK = EW_PAD // CHUNK
N_PAD = 10240
ROWS_PER_SUB = N_PAD // NS

BN = N_PAD // 5


def _k1_body(h_ref, r_ref, wq, bq, wk, bk, wv, bv, wrel, brel, attf,
             qp_o, k_o, v_o, rrel_o, qr_o):
    h = h_ref[...]
    q = jnp.dot(h, wq[...], preferred_element_type=jnp.float32) + bq[...]
    qp = q * attf[...]
    qp_o[...] = qp
    k_o[...] = jnp.dot(h, wk[...], preferred_element_type=jnp.float32) + bk[...]
    v_o[...] = jnp.dot(h, wv[...], preferred_element_type=jnp.float32) + bv[...]
    rrel = (jnp.dot(r_ref[...], wrel[...],
                    preferred_element_type=jnp.float32) + brel[...])
    qrs = []
    for h_i in range(HEADS):
        qp_h = qp[:, h_i * HD:(h_i + 1) * HD]
        rr_h = rrel[:, h_i * HD:(h_i + 1) * HD]
        qrs.append(jax.lax.dot_general(
            qp_h, rr_h, (((1,), (1,)), ((), ())),
            preferred_element_type=jnp.float32))
    qr_o[...] = jnp.concatenate(qrs, axis=1)

    @pl.when(pl.program_id(0) == 0)
    def _():
        rrel_o[...] = rrel


def _k1(H, R, Wq_w, Wq_b, Wk_w, Wk_b, Wv_w, Wv_b, Wrel_w, Wrel_b, att_full):
    full = pl.BlockSpec((DIM, DIM), lambda i: (0, 0))
    vec = pl.BlockSpec((1, DIM), lambda i: (0, 0))
    row = pl.BlockSpec((BN, DIM), lambda i: (i, 0))
    rfull = pl.BlockSpec((NREL, DIM), lambda i: (0, 0))
    out_sh = jax.ShapeDtypeStruct((N_PAD, DIM), jnp.float32)
    return pl.pallas_call(
        _k1_body,
        grid=(N_PAD // BN,),
        in_specs=[row, rfull, full, vec, full, vec, full, vec, full, vec, vec],
        out_specs=[row, row, row, rfull, row],
        out_shape=[out_sh, out_sh, out_sh,
                   jax.ShapeDtypeStruct((NREL, DIM), jnp.float32), out_sh],
    )(H, R, Wq_w, Wq_b.reshape(1, DIM), Wk_w, Wk_b.reshape(1, DIM),
      Wv_w, Wv_b.reshape(1, DIM), Wrel_w, Wrel_b.reshape(1, DIM),
      att_full.reshape(1, DIM))


def _unpack2(x):
    return plsc.unpack(plsc.bitcast(x, jnp.bfloat16),
                       format=plsc.PackFormat.INTERLEAVED,
                       preferred_element_type=jnp.float32)


def _k2_body(qp_hbm, kv_hbm, rrel_hbm, src_hbm, dst_hbm, et_hbm,
             attb_hbm, zeros_hbm, outp_hbm, attn_hbm,
             src2, dst2, et2, qp2, kv2,
             msg_rows, attn_v, rrel_v, attb_v, acc, sem_g, sem_i):
    cid = lax.axis_index("c")
    sid = lax.axis_index("s")
    wid = sid * NC + cid
    ebase = wid * EW_PAD

    pltpu.sync_copy(zeros_hbm.at[pl.ds(sid * ROWS_PER_SUB, ROWS_PER_SUB)],
                    acc.at[pl.ds(sid * ROWS_PER_SUB, ROWS_PER_SUB)])
    pltpu.sync_copy(attb_hbm, attb_v)
    pltpu.sync_copy(rrel_hbm, rrel_v)
    plsc.subcore_barrier()
    attb = attb_v[...]

    def fire_idx(ci, t):
        b = ebase + ci * CHUNK
        pltpu.async_copy(src_hbm.at[pl.ds(b, CHUNK)], src2.at[t], sem_i)
        pltpu.async_copy(dst_hbm.at[pl.ds(b, CHUNK)], dst2.at[t], sem_i)
        pltpu.async_copy(et_hbm.at[pl.ds(b, CHUNK)], et2.at[t], sem_i)

    def wait_idx(t):
        pltpu.make_async_copy(src_hbm.at[pl.ds(0, CHUNK)], src2.at[t],
                              sem_i).wait()
        pltpu.make_async_copy(dst_hbm.at[pl.ds(0, CHUNK)], dst2.at[t],
                              sem_i).wait()
        pltpu.make_async_copy(et_hbm.at[pl.ds(0, CHUNK)], et2.at[t],
                              sem_i).wait()

    def fire_gathers(t):
        pltpu.async_copy(qp_hbm.at[dst2.at[t]], qp2.at[t], sem_g)
        pltpu.async_copy(kv_hbm.at[src2.at[t]], kv2.at[t], sem_g)

    def wait_gathers(t):
        pltpu.make_async_copy(qp_hbm.at[dst2.at[t]], qp2.at[t], sem_g).wait()
        pltpu.make_async_copy(kv_hbm.at[src2.at[t]], kv2.at[t], sem_g).wait()

    def compute(ci, s):
        dst_v = dst2.at[s]
        et_ref = et2.at[s]
        qp_rows = qp2.at[s]
        kv_rows = kv2.at[s]

        def group_body(g):
            rows = jnp.arange(L, dtype=jnp.int32) + g * L
            et_g = et_ref[pl.ds(g * L, L)]
            et_par = jnp.equal(et_g & 1, 0)
            qr_col = (et_g >> 1) + (DIM // 2)
            wsum = jnp.zeros((L,), jnp.float32)
            ws = []
            for h in range(HEADS):
                qre, qro = _unpack2(plsc.load_gather(
                    qp_rows, [rows, qr_col + h * (NREL // 2)]))
                s = attb + jnp.where(et_par, qre, qro)
                for dp in range(HP):
                    cols = jnp.full((L,), h * HP + dp, dtype=jnp.int32)
                    qe, qo = _unpack2(plsc.load_gather(qp_rows, [rows, cols]))
                    ke, ko = _unpack2(plsc.load_gather(kv_rows, [rows, cols]))
                    s = s + qe * ke + qo * ko
                w = 1.0 / (1.0 + jnp.exp(-s))
                ws.append(w)
                wsum = wsum + w
            attn_v[pl.ds(ci * CHUNK + g * L, L)] = wsum * (1.0 / HEADS)
            for h in range(HEADS):
                for dp in range(HP):
                    cols = jnp.full((L,), h * HP + dp, dtype=jnp.int32)
                    ve, vo = _unpack2(
                        plsc.load_gather(kv_rows, [rows, cols + (DIM // 2)]))
                    re, ro = _unpack2(plsc.load_gather(rrel_v, [et_g, cols]))
                    ce = jnp.full((L,), h * HD + 2 * dp, dtype=jnp.int32)
                    plsc.store_scatter(msg_rows, [rows, ce], ws[h] * (ve + re))
                    plsc.store_scatter(msg_rows, [rows, ce + 1],
                                       ws[h] * (vo + ro))

        plsc.parallel_loop(0, CHUNK // L, 1, unroll=CHUNK // L)(group_body)
        pltpu.sync_copy(msg_rows, acc.at[dst_v], add=True)

    fire_idx(0, 0)
    wait_idx(0)
    fire_gathers(0)
    fire_idx(1, 1)

    def chunk_step(ci, _):
        s = ci & 1
        t = 1 - s
        wait_idx(t)
        fire_gathers(t)
        wait_gathers(s)
        compute(ci, s)
        fire_idx(jnp.minimum(ci + 2, NCHUNK - 1), s)
        return 0

    lax.fori_loop(0, NCHUNK, chunk_step, 0)
    wait_idx(0)
    wait_gathers(0)
    pltpu.sync_copy(attn_v, attn_hbm.at[pl.ds(ebase, EW_PAD)])
    plsc.subcore_barrier()
    pltpu.sync_copy(acc.at[pl.ds(sid * ROWS_PER_SUB, ROWS_PER_SUB)],
                    outp_hbm.at[cid, pl.ds(sid * ROWS_PER_SUB, ROWS_PER_SUB)])


def _k2(QTAB, KV, Rrel_p, src, dst, et, attb16, zeros):
    mesh = plsc.VectorSubcoreMesh(core_axis_name="c", subcore_axis_name="s")
    f = pl.kernel(
        _k2_body,
        out_type=(jax.ShapeDtypeStruct((NC, N_PAD, DIM), jnp.float32),
                  jax.ShapeDtypeStruct((E_PAD,), jnp.float32)),
        mesh=mesh,
        scratch_types=[
            pltpu.VMEM((2, CHUNK), jnp.int32),
            pltpu.VMEM((2, CHUNK), jnp.int32),
            pltpu.VMEM((2, CHUNK), jnp.int32),
            pltpu.VMEM((2, CHUNK, DIM), jnp.int32),
            pltpu.VMEM((2, CHUNK, DIM), jnp.int32),
            pltpu.VMEM((CHUNK, DIM), jnp.float32),
            pltpu.VMEM((EW_PAD,), jnp.float32),
            pltpu.VMEM((NREL, DIM // 2), jnp.int32),
            pltpu.VMEM((L,), jnp.float32),
            pltpu.VMEM_SHARED((N_PAD, DIM), jnp.float32),
            pltpu.SemaphoreType.DMA,
            pltpu.SemaphoreType.DMA,
        ],
        compiler_params=pltpu.CompilerParams(needs_layout_passes=False),
    )
    return f(QTAB, KV, Rrel_p, src, dst, et, attb16, zeros)


def _k3_body(op_ref, h_ref, w_ref, b_ref, g_ref, b2_ref, out_ref):
    o = op_ref[0] + op_ref[1]
    out = jnp.dot(o, w_ref[...], preferred_element_type=jnp.float32) + b_ref[...]
    act = jnp.where(out >= 0, out, 0.2 * out)
    x = h_ref[...] + act
    mu = jnp.mean(x, axis=1, keepdims=True)
    var = jnp.mean((x - mu) ** 2, axis=1, keepdims=True)
    out_ref[...] = (x - mu) / jnp.sqrt(var + 1e-5) * g_ref[...] + b2_ref[...]


def _k3(outp, H, out_w, out_b, ln_g, ln_b):
    full = pl.BlockSpec((DIM, DIM), lambda i: (0, 0))
    vec = pl.BlockSpec((1, DIM), lambda i: (0, 0))
    row = pl.BlockSpec((BN, DIM), lambda i: (i, 0))
    prow = pl.BlockSpec((NC, BN, DIM), lambda i: (0, i, 0))
    return pl.pallas_call(
        _k3_body,
        grid=(N_PAD // BN,),
        in_specs=[prow, row, full, vec, vec, vec],
        out_specs=row,
        out_shape=jax.ShapeDtypeStruct((N_PAD, DIM), jnp.float32),
    )(outp, H, out_w, out_b.reshape(1, DIM), ln_g.reshape(1, DIM),
      ln_b.reshape(1, DIM))


def kernel(H, R, edge_index, edge_type, Wq_w, Wq_b, Wk_w, Wk_b, Wv_w, Wv_b,
           att_w, att_b, Wrel_w, Wrel_b, out_w, out_b, ln_g, ln_b):
    att_full = jnp.tile(att_w[:, 0], HEADS)
    H_pad = jnp.pad(H, ((0, N_PAD - N), (0, 0)))
    Qp, K, V, Rrel, QR = _k1(H_pad, R, Wq_w, Wq_b, Wk_w, Wk_b, Wv_w, Wv_b,
                             Wrel_w, Wrel_b, att_full)

    def _pack(x):
        b = x.astype(jnp.bfloat16).reshape(x.shape[0], DIM // 2, 2)
        return jax.lax.bitcast_convert_type(b, jnp.int32)

    pad_spec = ((0, 0), (0, EW_PAD - EW))
    src_p = jnp.pad(edge_index[0].reshape(NW, EW), pad_spec).reshape(-1)
    dst_p = jnp.pad(edge_index[1].reshape(NW, EW), pad_spec,
                    constant_values=N_PAD - 1).reshape(-1)
    et_p = jnp.pad(edge_type.reshape(NW, EW), pad_spec).reshape(-1)

    attb16 = jnp.broadcast_to(att_b, (L,)).astype(jnp.float32)
    zeros = jnp.zeros((N_PAD, DIM), jnp.float32)
    KV = jnp.concatenate([_pack(K), _pack(V)], axis=1)
    QTAB = jnp.concatenate([_pack(Qp), _pack(QR)], axis=1)
    outp, attn_pad = _k2(QTAB, KV, _pack(Rrel), src_p, dst_p, et_p,
                         attb16, zeros)
    attn_mean = attn_pad.reshape(NW, EW_PAD)[:, :EW].reshape(E)
    h_out = _k3(outp, H_pad, out_w, out_b, ln_g, ln_b)[:N]
    return (h_out, attn_mean)


# speedup vs baseline: 17.6077x; 1.0291x over previous
import functools

import jax
import jax.numpy as jnp
from jax import lax
from jax.experimental import pallas as pl
from jax.experimental.pallas import tpu as pltpu
from jax.experimental.pallas import tpu_sc as plsc

N = 10000
E = 320000
DIM = 128
HEADS = 8
HD = DIM // HEADS
HP = HD // 2
NREL = 16

NC = 2
NS = 16
L = 16
NW = NC * NS
EW = E // NW
CHUNK = 48
EW_PAD = 10080
E_PAD = EW_PAD * NW
NCHUNK = EW_PAD // CHUNK
N_PAD = 10240
ROWS_PER_SUB = N_PAD // NS

BN = N_PAD // 5


def _k1_body(h_ref, r_ref, wq, bq, wk, bk, wv, bv, wrel, brel, attf,
             qp_o, k_o, v_o, rrel_o, qr_o):
    h = h_ref[...]
    q = jnp.dot(h, wq[...], preferred_element_type=jnp.float32) + bq[...]
    qp = q * attf[...]
    qp_o[...] = qp
    k_o[...] = jnp.dot(h, wk[...], preferred_element_type=jnp.float32) + bk[...]
    v_o[...] = jnp.dot(h, wv[...], preferred_element_type=jnp.float32) + bv[...]
    rrel = (jnp.dot(r_ref[...], wrel[...],
                    preferred_element_type=jnp.float32) + brel[...])
    qrs = []
    for h_i in range(HEADS):
        qp_h = qp[:, h_i * HD:(h_i + 1) * HD]
        rr_h = rrel[:, h_i * HD:(h_i + 1) * HD]
        qrs.append(jax.lax.dot_general(
            qp_h, rr_h, (((1,), (1,)), ((), ())),
            preferred_element_type=jnp.float32))
    qr_o[...] = jnp.concatenate(qrs, axis=1)

    @pl.when(pl.program_id(0) == 0)
    def _():
        rrel_o[...] = rrel


def _k1(H, R, Wq_w, Wq_b, Wk_w, Wk_b, Wv_w, Wv_b, Wrel_w, Wrel_b, att_full):
    full = pl.BlockSpec((DIM, DIM), lambda i: (0, 0))
    vec = pl.BlockSpec((1, DIM), lambda i: (0, 0))
    row = pl.BlockSpec((BN, DIM), lambda i: (i, 0))
    rfull = pl.BlockSpec((NREL, DIM), lambda i: (0, 0))
    out_sh = jax.ShapeDtypeStruct((N_PAD, DIM), jnp.float32)
    return pl.pallas_call(
        _k1_body,
        grid=(N_PAD // BN,),
        in_specs=[row, rfull, full, vec, full, vec, full, vec, full, vec, vec],
        out_specs=[row, row, row, rfull, row],
        out_shape=[out_sh, out_sh, out_sh,
                   jax.ShapeDtypeStruct((NREL, DIM), jnp.float32), out_sh],
    )(H, R, Wq_w, Wq_b.reshape(1, DIM), Wk_w, Wk_b.reshape(1, DIM),
      Wv_w, Wv_b.reshape(1, DIM), Wrel_w, Wrel_b.reshape(1, DIM),
      att_full.reshape(1, DIM))


def _unpack2(x):
    return plsc.unpack(plsc.bitcast(x, jnp.bfloat16),
                       format=plsc.PackFormat.INTERLEAVED,
                       preferred_element_type=jnp.float32)


def _k2_body(qp_hbm, kv_hbm, rrel_hbm, src_hbm, dst_hbm, et_hbm,
             attb_hbm, zeros_hbm, outp_hbm, attn_hbm,
             src2, dst2, et2, qp2, kv2,
             msg_rows, attn_v, rrel_v, attb_v, acc, sem_g, sem_i):
    cid = lax.axis_index("c")
    sid = lax.axis_index("s")
    wid = sid * NC + cid
    ebase = wid * EW_PAD

    pltpu.sync_copy(zeros_hbm.at[pl.ds(sid * ROWS_PER_SUB, ROWS_PER_SUB)],
                    acc.at[pl.ds(sid * ROWS_PER_SUB, ROWS_PER_SUB)])
    pltpu.sync_copy(attb_hbm, attb_v)
    pltpu.sync_copy(rrel_hbm, rrel_v)
    plsc.subcore_barrier()
    attb = attb_v[...]

    def fire_idx(ci, t):
        b = ebase + ci * CHUNK
        pltpu.async_copy(src_hbm.at[pl.ds(b, CHUNK)], src2.at[t], sem_i)
        pltpu.async_copy(dst_hbm.at[pl.ds(b, CHUNK)], dst2.at[t], sem_i)
        pltpu.async_copy(et_hbm.at[pl.ds(b, CHUNK)], et2.at[t], sem_i)

    def wait_idx(t):
        pltpu.make_async_copy(src_hbm.at[pl.ds(0, CHUNK)], src2.at[t],
                              sem_i).wait()
        pltpu.make_async_copy(dst_hbm.at[pl.ds(0, CHUNK)], dst2.at[t],
                              sem_i).wait()
        pltpu.make_async_copy(et_hbm.at[pl.ds(0, CHUNK)], et2.at[t],
                              sem_i).wait()

    def fire_gathers(t):
        pltpu.async_copy(qp_hbm.at[dst2.at[t]], qp2.at[t], sem_g)
        pltpu.async_copy(kv_hbm.at[src2.at[t]], kv2.at[t], sem_g)

    def wait_gathers(t):
        pltpu.make_async_copy(qp_hbm.at[dst2.at[t]], qp2.at[t], sem_g).wait()
        pltpu.make_async_copy(kv_hbm.at[src2.at[t]], kv2.at[t], sem_g).wait()

    def compute(ci, s):
        dst_v = dst2.at[s]
        et_ref = et2.at[s]
        qp_rows = qp2.at[s]
        kv_rows = kv2.at[s]

        def group_body(g):
            lane = jnp.arange(L, dtype=jnp.int32)
            rows = lane + g * L
            et_g = et_ref[pl.ds(g * L, L)]
            et_par = jnp.equal(et_g & 1, 0)
            qr_col = (et_g >> 1) + (DIM // 2)
            rot = [(lane + dp) & (HP - 1) for dp in range(HP)]
            wsum = jnp.zeros((L,), jnp.float32)
            ws = []
            for h in range(HEADS):
                qre, qro = _unpack2(plsc.load_gather(
                    qp_rows, [rows, qr_col + h * (NREL // 2)]))
                s = attb + jnp.where(et_par, qre, qro)
                for dp in range(HP):
                    cols = rot[dp] + h * HP
                    qe, qo = _unpack2(plsc.load_gather(qp_rows, [rows, cols]))
                    ke, ko = _unpack2(plsc.load_gather(kv_rows, [rows, cols]))
                    s = s + qe * ke + qo * ko
                w = 1.0 / (1.0 + jnp.exp(-s))
                ws.append(w)
                wsum = wsum + w
            attn_v[pl.ds(ci * CHUNK + g * L, L)] = wsum * (1.0 / HEADS)
            for h in range(HEADS):
                for dp in range(HP):
                    cols = rot[dp] + h * HP
                    ve, vo = _unpack2(
                        plsc.load_gather(kv_rows, [rows, cols + (DIM // 2)]))
                    re, ro = _unpack2(plsc.load_gather(rrel_v, [et_g, cols]))
                    ce = rot[dp] * 2 + h * HD
                    plsc.store_scatter(msg_rows, [rows, ce], ws[h] * (ve + re))
                    plsc.store_scatter(msg_rows, [rows, ce + 1],
                                       ws[h] * (vo + ro))

        plsc.parallel_loop(0, CHUNK // L, 1, unroll=CHUNK // L)(group_body)
        pltpu.sync_copy(msg_rows, acc.at[dst_v], add=True)

    fire_idx(0, 0)
    wait_idx(0)
    fire_gathers(0)
    fire_idx(1, 1)

    def chunk_step(ci, _):
        s = ci & 1
        t = 1 - s
        wait_idx(t)
        fire_gathers(t)
        wait_gathers(s)
        compute(ci, s)
        fire_idx(jnp.minimum(ci + 2, NCHUNK - 1), s)
        return 0

    lax.fori_loop(0, NCHUNK, chunk_step, 0)
    wait_idx(0)
    wait_gathers(0)
    pltpu.sync_copy(attn_v, attn_hbm.at[pl.ds(ebase, EW_PAD)])
    plsc.subcore_barrier()
    pltpu.sync_copy(acc.at[pl.ds(sid * ROWS_PER_SUB, ROWS_PER_SUB)],
                    outp_hbm.at[cid, pl.ds(sid * ROWS_PER_SUB, ROWS_PER_SUB)])


def _k2(QTAB, KV, Rrel_p, src, dst, et, attb16, zeros):
    mesh = plsc.VectorSubcoreMesh(core_axis_name="c", subcore_axis_name="s")
    f = pl.kernel(
        _k2_body,
        out_type=(jax.ShapeDtypeStruct((NC, N_PAD, DIM), jnp.float32),
                  jax.ShapeDtypeStruct((E_PAD,), jnp.float32)),
        mesh=mesh,
        scratch_types=[
            pltpu.VMEM((2, CHUNK), jnp.int32),
            pltpu.VMEM((2, CHUNK), jnp.int32),
            pltpu.VMEM((2, CHUNK), jnp.int32),
            pltpu.VMEM((2, CHUNK, DIM), jnp.int32),
            pltpu.VMEM((2, CHUNK, DIM), jnp.int32),
            pltpu.VMEM((CHUNK, DIM), jnp.float32),
            pltpu.VMEM((EW_PAD,), jnp.float32),
            pltpu.VMEM((NREL, DIM // 2), jnp.int32),
            pltpu.VMEM((L,), jnp.float32),
            pltpu.VMEM_SHARED((N_PAD, DIM), jnp.float32),
            pltpu.SemaphoreType.DMA,
            pltpu.SemaphoreType.DMA,
        ],
        compiler_params=pltpu.CompilerParams(needs_layout_passes=False),
    )
    return f(QTAB, KV, Rrel_p, src, dst, et, attb16, zeros)


def _k3_body(op_ref, h_ref, w_ref, b_ref, g_ref, b2_ref, out_ref):
    o = op_ref[0] + op_ref[1]
    out = jnp.dot(o, w_ref[...], preferred_element_type=jnp.float32) + b_ref[...]
    act = jnp.where(out >= 0, out, 0.2 * out)
    x = h_ref[...] + act
    mu = jnp.mean(x, axis=1, keepdims=True)
    var = jnp.mean((x - mu) ** 2, axis=1, keepdims=True)
    out_ref[...] = (x - mu) / jnp.sqrt(var + 1e-5) * g_ref[...] + b2_ref[...]


def _k3(outp, H, out_w, out_b, ln_g, ln_b):
    full = pl.BlockSpec((DIM, DIM), lambda i: (0, 0))
    vec = pl.BlockSpec((1, DIM), lambda i: (0, 0))
    row = pl.BlockSpec((BN, DIM), lambda i: (i, 0))
    prow = pl.BlockSpec((NC, BN, DIM), lambda i: (0, i, 0))
    return pl.pallas_call(
        _k3_body,
        grid=(N_PAD // BN,),
        in_specs=[prow, row, full, vec, vec, vec],
        out_specs=row,
        out_shape=jax.ShapeDtypeStruct((N_PAD, DIM), jnp.float32),
    )(outp, H, out_w, out_b.reshape(1, DIM), ln_g.reshape(1, DIM),
      ln_b.reshape(1, DIM))


def kernel(H, R, edge_index, edge_type, Wq_w, Wq_b, Wk_w, Wk_b, Wv_w, Wv_b,
           att_w, att_b, Wrel_w, Wrel_b, out_w, out_b, ln_g, ln_b):
    att_full = jnp.tile(att_w[:, 0], HEADS)
    H_pad = jnp.pad(H, ((0, N_PAD - N), (0, 0)))
    Qp, K, V, Rrel, QR = _k1(H_pad, R, Wq_w, Wq_b, Wk_w, Wk_b, Wv_w, Wv_b,
                             Wrel_w, Wrel_b, att_full)

    def _pack(x):
        b = x.astype(jnp.bfloat16).reshape(x.shape[0], DIM // 2, 2)
        return jax.lax.bitcast_convert_type(b, jnp.int32)

    pad_spec = ((0, 0), (0, EW_PAD - EW))
    src_p = jnp.pad(edge_index[0].reshape(NW, EW), pad_spec).reshape(-1)
    dst_p = jnp.pad(edge_index[1].reshape(NW, EW), pad_spec,
                    constant_values=N_PAD - 1).reshape(-1)
    et_p = jnp.pad(edge_type.reshape(NW, EW), pad_spec).reshape(-1)

    attb16 = jnp.broadcast_to(att_b, (L,)).astype(jnp.float32)
    zeros = jnp.zeros((N_PAD, DIM), jnp.float32)
    KV = jnp.concatenate([_pack(K), _pack(V)], axis=1)
    QTAB = jnp.concatenate([_pack(Qp), _pack(QR)], axis=1)
    outp, attn_pad = _k2(QTAB, KV, _pack(Rrel), src_p, dst_p, et_p,
                         attb16, zeros)
    attn_mean = attn_pad.reshape(NW, EW_PAD)[:, :EW].reshape(E)
    h_out = _k3(outp, H_pad, out_w, out_b, ln_g, ln_b)[:N]
    return (h_out, attn_mean)
